# baseline (speedup 1.0000x reference)
import functools

import jax
import jax.numpy as jnp
from jax.experimental import pallas as pl
from jax.experimental.pallas import tpu as pltpu

P = 10000
U = 10000
D = 128
BM = 400


def _mm_body(a_ref, b_ref, o_ref):
    o_ref[...] = jax.lax.dot_general(
        a_ref[...], b_ref[...],
        dimension_numbers=(((1,), (0,)), ((), ())),
        preferred_element_type=jnp.float32,
    )


def _mm(a, b):
    m, k = a.shape
    _, n = b.shape
    return pl.pallas_call(
        _mm_body,
        grid=(m // BM,),
        in_specs=[
            pl.BlockSpec((BM, k), lambda i: (i, 0)),
            pl.BlockSpec((k, n), lambda i: (0, 0)),
        ],
        out_specs=pl.BlockSpec((BM, n), lambda i: (i, 0)),
        out_shape=jax.ShapeDtypeStruct((m, n), jnp.float32),
        compiler_params=pltpu.CompilerParams(
            dimension_semantics=("arbitrary",),
        ),
    )(a, b)


@jax.jit
def _fused(pois_embs, HG_up, HG_pu):
    tmp = _mm(HG_up, pois_embs)
    return _mm(HG_pu, tmp)


def kernel(pois_embs, pad_all_train_sessions, HG_up, HG_pu):
    del pad_all_train_sessions
    return _fused(pois_embs, HG_up, HG_pu)


# speedup vs baseline: 1.0227x; 1.0227x over previous
import jax
import jax.numpy as jnp
from jax.experimental import pallas as pl
from jax.experimental.pallas import tpu as pltpu

P = 10000
U = 10000
D = 128
BM = 256
NB = (U + BM - 1) // BM
REM = U - (NB - 1) * BM


def _dot(a, b):
    return jax.lax.dot_general(
        a, b, dimension_numbers=(((1,), (0,)), ((), ())),
        preferred_element_type=jnp.float32,
    )


def _fused_body(up_ref, pu_ref, pe_ref, out_ref, tmp_ref):
    i = pl.program_id(0)

    @pl.when(i < NB)
    def _phase1():
        blk = _dot(up_ref[...], pe_ref[...])

        @pl.when(i < NB - 1)
        def _store_full():
            tmp_ref[i % 2] = blk

        @pl.when(i == NB - 1)
        def _store_masked():
            rows = jax.lax.broadcasted_iota(jnp.int32, (BM, D), 0)
            tmp_ref[i % 2] = jnp.where(rows < REM, blk, 0.0)

    @pl.when(i == 1)
    def _phase2_first():
        out_ref[...] = _dot(pu_ref[...], tmp_ref[(i - 1) % 2])

    @pl.when((i > 1) & (i < NB))
    def _phase2_full():
        out_ref[...] += _dot(pu_ref[...], tmp_ref[(i - 1) % 2])

    @pl.when(i == NB)
    def _phase2_partial():
        cols = jax.lax.broadcasted_iota(jnp.int32, (P, BM), 1)
        pu_blk = jnp.where(cols < REM, pu_ref[...], 0.0)
        out_ref[...] += _dot(pu_blk, tmp_ref[(i - 1) % 2])


@jax.jit
def _fused(pois_embs, HG_up, HG_pu):
    return pl.pallas_call(
        _fused_body,
        grid=(NB + 1,),
        in_specs=[
            pl.BlockSpec((BM, P), lambda i: (jnp.minimum(i, NB - 1), 0)),
            pl.BlockSpec((P, BM), lambda i: (0, jnp.clip(i - 1, 0, NB - 1))),
            pl.BlockSpec((P, D), lambda i: (0, 0)),
        ],
        out_specs=pl.BlockSpec((P, D), lambda i: (0, 0)),
        out_shape=jax.ShapeDtypeStruct((P, D), jnp.float32),
        scratch_shapes=[pltpu.VMEM((2, BM, D), jnp.float32)],
        compiler_params=pltpu.CompilerParams(
            dimension_semantics=("arbitrary",),
        ),
    )(HG_up, HG_pu, pois_embs)


def kernel(pois_embs, pad_all_train_sessions, HG_up, HG_pu):
    del pad_all_train_sessions
    return _fused(pois_embs, HG_up, HG_pu)
